# baseline (speedup 1.0000x reference)
import jax
import jax.numpy as jnp
from jax import lax
from jax.experimental import pallas as pl
from jax.experimental.pallas import tpu as pltpu
from jax.experimental.pallas import tpu_sc as plsc

N_NODES = 10000
N_EDGES = 160000
D_IN = 256
D_H = 512
D_OUT = 64
D_E = 16
D_P = 32

NC = 2
NS = 16
CHUNK = 128
NP = 10240
EP = 163840
CHUNKS_PER_TILE = EP // NS // CHUNK
ROWS_PER_TILE = NP // NS

_MESH = plsc.VectorSubcoreMesh(
    core_axis_name="c", subcore_axis_name="s", num_cores=NC, num_subcores=NS)


_CHUNKS_PER_WORKER = EP // CHUNK // (NC * NS)


def _sc1_body(xs, srcp2, dstp, pay, z128, z32, outx, outp,
              accx, accp, sslot, dslot, rows, pv, gsem, isem, psem):
    c = lax.axis_index("c")
    s = lax.axis_index("s")
    r0 = s * ROWS_PER_TILE
    pltpu.sync_copy(z128.at[pl.ds(r0, ROWS_PER_TILE)],
                    accx.at[pl.ds(r0, ROWS_PER_TILE)])
    pltpu.sync_copy(z32.at[pl.ds(r0, ROWS_PER_TILE)],
                    accp.at[pl.ds(r0, ROWS_PER_TILE)])
    plsc.subcore_barrier()

    srow0 = c * (EP // CHUNK) + s * CHUNKS_PER_TILE
    drow0 = s * CHUNKS_PER_TILE

    pltpu.sync_copy(srcp2.at[srow0], sslot.at[0])
    pltpu.sync_copy(dstp.at[drow0], dslot.at[0])
    pltpu.async_copy(srcp2.at[srow0 + 1], sslot.at[1], isem)
    pltpu.async_copy(dstp.at[drow0 + 1], dslot.at[1], isem)
    pltpu.async_copy(xs.at[sslot.at[0]], rows.at[0], gsem)
    pltpu.sync_copy(pay.at[pl.ds((drow0 + c) * CHUNK, CHUNK)], pv.at[0])
    pltpu.async_copy(pay.at[pl.ds((drow0 + c + 2) * CHUNK, CHUNK)], pv.at[1],
                     psem)

    def chunk_body(i, carry):
        b = lax.rem(i, 2)
        nb = 1 - b

        @pl.when(i < CHUNKS_PER_TILE - 1)
        def _():
            pltpu.make_async_copy(srcp2.at[srow0], sslot.at[nb], isem).wait()
            pltpu.make_async_copy(dstp.at[drow0], dslot.at[nb], isem).wait()
            pltpu.async_copy(xs.at[sslot.at[nb]], rows.at[nb], gsem)

        pltpu.make_async_copy(xs.at[pl.ds(0, CHUNK)], rows.at[b], gsem).wait()
        pltpu.sync_copy(rows.at[b], accx.at[dslot.at[b]], add=True)

        @pl.when(b == c)
        def _():
            k = (i - c) // 2
            pb = lax.rem(k, 2)

            @pl.when(k >= 1)
            def _():
                pltpu.make_async_copy(pay.at[pl.ds(0, CHUNK)], pv.at[pb],
                                      psem).wait()

            pltpu.sync_copy(pv.at[pb], accp.at[dslot.at[b]], add=True)

            @pl.when(i + 4 < CHUNKS_PER_TILE)
            def _():
                pltpu.async_copy(pay.at[pl.ds((drow0 + i + 4) * CHUNK, CHUNK)],
                                 pv.at[pb], psem)

        @pl.when(i < CHUNKS_PER_TILE - 2)
        def _():
            pltpu.async_copy(srcp2.at[srow0 + i + 2], sslot.at[b], isem)
            pltpu.async_copy(dstp.at[drow0 + i + 2], dslot.at[b], isem)

        return carry

    lax.fori_loop(0, CHUNKS_PER_TILE, chunk_body, 0)
    plsc.subcore_barrier()
    pltpu.sync_copy(accx.at[pl.ds(r0, ROWS_PER_TILE)],
                    outx.at[pl.ds(c * NP + r0, ROWS_PER_TILE)])
    pltpu.sync_copy(accp.at[pl.ds(r0, ROWS_PER_TILE)],
                    outp.at[pl.ds(c * NP + r0, ROWS_PER_TILE)])


_sc1 = pl.kernel(
    _sc1_body,
    out_type=(jax.ShapeDtypeStruct((2 * NP, 128), jnp.float32),
              jax.ShapeDtypeStruct((2 * NP, D_P), jnp.bfloat16)),
    mesh=_MESH,
    scratch_types=[
        pltpu.VMEM_SHARED((NP, 128), jnp.float32),
        pltpu.VMEM_SHARED((NP, D_P), jnp.bfloat16),
        pltpu.VMEM((2, CHUNK), jnp.int32),
        pltpu.VMEM((2, CHUNK), jnp.int32),
        pltpu.VMEM((2, CHUNK, 128), jnp.float32),
        pltpu.VMEM((2, CHUNK, D_P), jnp.bfloat16),
        pltpu.SemaphoreType.DMA,
        pltpu.SemaphoreType.DMA,
        pltpu.SemaphoreType.DMA,
    ],
    compiler_params=pltpu.CompilerParams(use_tc_tiling_on_sc=False),
)


def _sc2_body(z, srcp, dstp, z128, out2, acc2, sslot, dslot, rows, gsem, isem):
    c = lax.axis_index("c")
    s = lax.axis_index("s")
    r0 = s * ROWS_PER_TILE
    pltpu.sync_copy(z128.at[pl.ds(r0, ROWS_PER_TILE)],
                    acc2.at[pl.ds(r0, ROWS_PER_TILE)])
    plsc.subcore_barrier()

    wid = s * NC + c
    ch0 = wid * _CHUNKS_PER_WORKER

    pltpu.sync_copy(srcp.at[ch0], sslot.at[0])
    pltpu.sync_copy(dstp.at[ch0], dslot.at[0])
    pltpu.async_copy(srcp.at[ch0 + 1], sslot.at[1], isem)
    pltpu.async_copy(dstp.at[ch0 + 1], dslot.at[1], isem)
    pltpu.async_copy(z.at[sslot.at[0]], rows.at[0], gsem)

    def chunk_body(i, carry):
        b = lax.rem(i, 2)
        nb = 1 - b

        @pl.when(i < _CHUNKS_PER_WORKER - 1)
        def _():
            pltpu.make_async_copy(srcp.at[ch0], sslot.at[nb], isem).wait()
            pltpu.make_async_copy(dstp.at[ch0], dslot.at[nb], isem).wait()
            pltpu.async_copy(z.at[sslot.at[nb]], rows.at[nb], gsem)

        pltpu.make_async_copy(z.at[pl.ds(0, CHUNK)], rows.at[b], gsem).wait()
        pltpu.sync_copy(rows.at[b], acc2.at[dslot.at[b]], add=True)

        @pl.when(i < _CHUNKS_PER_WORKER - 2)
        def _():
            pltpu.async_copy(srcp.at[ch0 + i + 2], sslot.at[b], isem)
            pltpu.async_copy(dstp.at[ch0 + i + 2], dslot.at[b], isem)

        return carry

    lax.fori_loop(0, _CHUNKS_PER_WORKER, chunk_body, 0)
    plsc.subcore_barrier()
    pltpu.sync_copy(acc2.at[pl.ds(r0, ROWS_PER_TILE)],
                    out2.at[pl.ds(c * NP + r0, ROWS_PER_TILE)])


_sc2 = pl.kernel(
    _sc2_body,
    out_type=jax.ShapeDtypeStruct((2 * NP, 128), jnp.float32),
    mesh=_MESH,
    scratch_types=[
        pltpu.VMEM_SHARED((NP, 128), jnp.float32),
        pltpu.VMEM((2, CHUNK), jnp.int32),
        pltpu.VMEM((2, CHUNK), jnp.int32),
        pltpu.VMEM((2, CHUNK, 128), jnp.float32),
        pltpu.SemaphoreType.DMA,
        pltpu.SemaphoreType.DMA,
    ],
)


BM = 512
_PREC = lax.Precision.HIGHEST


def _tc1_body(x_r, a0_r, a1_r, p0_r, p1_r, wl1a_r, wl1b_r, wr1_r, we_r,
              bl1_r, be_r, wl2_r, h_r, z_r):
    p = p0_r[:] + p1_r[:]
    deg = p[:, 16:17]
    inv = 1.0 / jnp.maximum(deg, 1.0)
    acc = jnp.dot(a0_r[:] * inv, wl1a_r[:], precision=_PREC)
    acc = acc + jnp.dot(a1_r[:] * inv, wl1b_r[:], precision=_PREC)
    acc = acc + jnp.dot(x_r[:], wr1_r[:], precision=_PREC)
    acc = acc + jnp.dot(p[:, :D_E], we_r[:], precision=_PREC)
    acc = acc + bl1_r[:] + deg * be_r[:]
    h = jnp.maximum(acc, 0.0)
    h_r[:] = h
    zd = jnp.dot(h, wl2_r[:], precision=_PREC)
    z_r[:] = jnp.concatenate(
        [zd, jnp.zeros((zd.shape[0], 128 - D_OUT), jnp.float32)], axis=1)


def _tc1(xp, a0, a1, p0, p1, wl1a, wl1b, wr1, we, bl1, be, wl2):
    grid = (NP // BM,)
    return pl.pallas_call(
        _tc1_body,
        grid=grid,
        in_specs=[
            pl.BlockSpec((BM, D_IN), lambda i: (i, 0)),
            pl.BlockSpec((BM, 128), lambda i: (i, 0)),
            pl.BlockSpec((BM, 128), lambda i: (i, 0)),
            pl.BlockSpec((BM, D_P), lambda i: (i, 0)),
            pl.BlockSpec((BM, D_P), lambda i: (i, 0)),
            pl.BlockSpec((128, D_H), lambda i: (0, 0)),
            pl.BlockSpec((128, D_H), lambda i: (0, 0)),
            pl.BlockSpec((D_IN, D_H), lambda i: (0, 0)),
            pl.BlockSpec((D_E, D_H), lambda i: (0, 0)),
            pl.BlockSpec((1, D_H), lambda i: (0, 0)),
            pl.BlockSpec((1, D_H), lambda i: (0, 0)),
            pl.BlockSpec((D_H, D_OUT), lambda i: (0, 0)),
        ],
        out_specs=[
            pl.BlockSpec((BM, D_H), lambda i: (i, 0)),
            pl.BlockSpec((BM, 128), lambda i: (i, 0)),
        ],
        out_shape=[
            jax.ShapeDtypeStruct((NP, D_H), jnp.float32),
            jax.ShapeDtypeStruct((NP, 128), jnp.float32),
        ],
    )(xp, a0, a1, p0, p1, wl1a, wl1b, wr1, we, bl1, be, wl2)


def _tc2_body(h_r, b0_r, b1_r, p0_r, p1_r, wr2_r, bl2_r, o_r):
    p = p0_r[:] + p1_r[:]
    inv = 1.0 / jnp.maximum(p[:, 16:17], 1.0)
    o = (b0_r[:, :D_OUT] + b1_r[:, :D_OUT]) * inv
    o = o + jnp.dot(h_r[:], wr2_r[:], precision=_PREC)
    o_r[:] = o + bl2_r[:]


def _tc2(h, b0, b1, p0, p1, wr2, bl2):
    grid = (NP // BM,)
    return pl.pallas_call(
        _tc2_body,
        grid=grid,
        in_specs=[
            pl.BlockSpec((BM, D_H), lambda i: (i, 0)),
            pl.BlockSpec((BM, 128), lambda i: (i, 0)),
            pl.BlockSpec((BM, 128), lambda i: (i, 0)),
            pl.BlockSpec((BM, D_P), lambda i: (i, 0)),
            pl.BlockSpec((BM, D_P), lambda i: (i, 0)),
            pl.BlockSpec((D_H, D_OUT), lambda i: (0, 0)),
            pl.BlockSpec((1, D_OUT), lambda i: (0, 0)),
        ],
        out_specs=pl.BlockSpec((BM, D_OUT), lambda i: (i, 0)),
        out_shape=jax.ShapeDtypeStruct((NP, D_OUT), jnp.float32),
    )(h, b0, b1, p0, p1, wr2, bl2)


@jax.jit
def kernel(x, edge_index, edge_attr, Wl1, bl1, Wr1, We, be, Wl2, bl2, Wr2):
    src = edge_index[0]
    dst = edge_index[1]

    xp = jnp.pad(x, ((0, NP - N_NODES), (0, 0)))
    xs = jnp.concatenate([xp[:, :128], xp[:, 128:]], axis=0)
    srcp = jnp.pad(src, (0, EP - N_EDGES)).reshape(EP // CHUNK, CHUNK)
    srcp2 = jnp.concatenate([srcp, srcp + NP], axis=0)
    dstp = jnp.pad(dst, (0, EP - N_EDGES),
                   constant_values=N_NODES).reshape(EP // CHUNK, CHUNK)
    pay = jnp.concatenate(
        [edge_attr, jnp.ones((N_EDGES, 1), jnp.float32),
         jnp.zeros((N_EDGES, D_P - D_E - 1), jnp.float32)], axis=1)
    pay = jnp.pad(pay, ((0, EP - N_EDGES), (0, 0))).astype(jnp.bfloat16)
    z128 = jnp.zeros((NP, 128), jnp.float32)
    z32 = jnp.zeros((NP, D_P), jnp.bfloat16)

    aggx, aggp = _sc1(xs, srcp2, dstp, pay, z128, z32)
    a0, a1 = aggx[:NP], aggx[NP:]
    p0 = aggp[:NP].astype(jnp.float32)
    p1 = aggp[NP:].astype(jnp.float32)

    h, z = _tc1(xp, a0, a1, p0, p1,
                Wl1[:128], Wl1[128:], Wr1, We,
                bl1.reshape(1, D_H), be.reshape(1, D_H), Wl2)

    agg2 = _sc2(z, srcp, dstp, z128)
    out = _tc2(h, agg2[:NP], agg2[NP:], p0, p1, Wr2, bl2.reshape(1, D_OUT))
    return out[:N_NODES]


# speedup vs baseline: 1.1247x; 1.1247x over previous
import jax
import jax.numpy as jnp
from jax import lax
from jax.experimental import pallas as pl
from jax.experimental.pallas import tpu as pltpu
from jax.experimental.pallas import tpu_sc as plsc

N_NODES = 10000
N_EDGES = 160000
D_IN = 256
D_H = 512
D_OUT = 64
D_E = 16
D_P = 32

NC = 2
NS = 16
CHUNK = 128
NP = 10240
EP = 163840
CHUNKS_PER_TILE = EP // NS // CHUNK
ROWS_PER_TILE = NP // NS

_MESH = plsc.VectorSubcoreMesh(
    core_axis_name="c", subcore_axis_name="s", num_cores=NC, num_subcores=NS)


_CHUNKS_PER_WORKER = EP // CHUNK // (NC * NS)


def _sc1_body(xs, srcp2, dstp, pay, z128, z32, outx, outp,
              accx, accp, sslot, dslot, rows, pv, gsem, isem, psem):
    c = lax.axis_index("c")
    s = lax.axis_index("s")
    r0 = s * ROWS_PER_TILE
    pltpu.sync_copy(z128.at[pl.ds(r0, ROWS_PER_TILE)],
                    accx.at[pl.ds(r0, ROWS_PER_TILE)])
    pltpu.sync_copy(z32.at[pl.ds(r0, ROWS_PER_TILE)],
                    accp.at[pl.ds(r0, ROWS_PER_TILE)])
    plsc.subcore_barrier()

    srow0 = c * (EP // CHUNK) + s * CHUNKS_PER_TILE
    drow0 = s * CHUNKS_PER_TILE

    pltpu.sync_copy(srcp2.at[srow0], sslot.at[0])
    pltpu.sync_copy(dstp.at[drow0], dslot.at[0])
    pltpu.async_copy(srcp2.at[srow0 + 1], sslot.at[1], isem)
    pltpu.async_copy(dstp.at[drow0 + 1], dslot.at[1], isem)
    pltpu.async_copy(xs.at[sslot.at[0]], rows.at[0], gsem)
    pltpu.sync_copy(pay.at[pl.ds((drow0 + c) * CHUNK, CHUNK)], pv.at[0])
    pltpu.async_copy(pay.at[pl.ds((drow0 + c + 2) * CHUNK, CHUNK)], pv.at[1],
                     psem)

    def chunk_body(i, carry):
        b = lax.rem(i, 2)
        nb = 1 - b

        @pl.when(i < CHUNKS_PER_TILE - 1)
        def _():
            pltpu.make_async_copy(srcp2.at[srow0], sslot.at[nb], isem).wait()
            pltpu.make_async_copy(dstp.at[drow0], dslot.at[nb], isem).wait()
            pltpu.async_copy(xs.at[sslot.at[nb]], rows.at[nb], gsem)

        pltpu.make_async_copy(xs.at[pl.ds(0, CHUNK)], rows.at[b], gsem).wait()
        pltpu.sync_copy(rows.at[b], accx.at[dslot.at[b]], add=True)

        @pl.when(b == c)
        def _():
            k = (i - c) // 2
            pb = lax.rem(k, 2)

            @pl.when(k >= 1)
            def _():
                pltpu.make_async_copy(pay.at[pl.ds(0, CHUNK)], pv.at[pb],
                                      psem).wait()

            pltpu.sync_copy(pv.at[pb], accp.at[dslot.at[b]], add=True)

            @pl.when(i + 4 < CHUNKS_PER_TILE)
            def _():
                pltpu.async_copy(pay.at[pl.ds((drow0 + i + 4) * CHUNK, CHUNK)],
                                 pv.at[pb], psem)

        @pl.when(i < CHUNKS_PER_TILE - 2)
        def _():
            pltpu.async_copy(srcp2.at[srow0 + i + 2], sslot.at[b], isem)
            pltpu.async_copy(dstp.at[drow0 + i + 2], dslot.at[b], isem)

        return carry

    lax.fori_loop(0, CHUNKS_PER_TILE, chunk_body, 0)
    plsc.subcore_barrier()
    pltpu.sync_copy(accx.at[pl.ds(r0, ROWS_PER_TILE)],
                    outx.at[pl.ds(c * NP + r0, ROWS_PER_TILE)])
    pltpu.sync_copy(accp.at[pl.ds(r0, ROWS_PER_TILE)],
                    outp.at[pl.ds(c * NP + r0, ROWS_PER_TILE)])


_sc1 = pl.kernel(
    _sc1_body,
    out_type=(jax.ShapeDtypeStruct((2 * NP, 128), jnp.float32),
              jax.ShapeDtypeStruct((2 * NP, D_P), jnp.bfloat16)),
    mesh=_MESH,
    scratch_types=[
        pltpu.VMEM_SHARED((NP, 128), jnp.float32),
        pltpu.VMEM_SHARED((NP, D_P), jnp.bfloat16),
        pltpu.VMEM((2, CHUNK), jnp.int32),
        pltpu.VMEM((2, CHUNK), jnp.int32),
        pltpu.VMEM((2, CHUNK, 128), jnp.float32),
        pltpu.VMEM((2, CHUNK, D_P), jnp.bfloat16),
        pltpu.SemaphoreType.DMA,
        pltpu.SemaphoreType.DMA,
        pltpu.SemaphoreType.DMA,
    ],
    compiler_params=pltpu.CompilerParams(use_tc_tiling_on_sc=False),
)


def _sc2_body(z, srcp, dstp, z64, out2, acc2, sslot, dslot, rows, gsem, isem):
    c = lax.axis_index("c")
    s = lax.axis_index("s")
    r0 = s * ROWS_PER_TILE
    pltpu.sync_copy(z64.at[pl.ds(r0, ROWS_PER_TILE)],
                    acc2.at[pl.ds(r0, ROWS_PER_TILE)])
    plsc.subcore_barrier()

    wid = s * NC + c
    ch0 = wid * _CHUNKS_PER_WORKER

    pltpu.sync_copy(srcp.at[ch0], sslot.at[0])
    pltpu.sync_copy(dstp.at[ch0], dslot.at[0])
    pltpu.async_copy(srcp.at[ch0 + 1], sslot.at[1], isem)
    pltpu.async_copy(dstp.at[ch0 + 1], dslot.at[1], isem)
    pltpu.async_copy(z.at[sslot.at[0]], rows.at[0], gsem)

    def chunk_body(i, carry):
        b = lax.rem(i, 2)
        nb = 1 - b

        @pl.when(i < _CHUNKS_PER_WORKER - 1)
        def _():
            pltpu.make_async_copy(srcp.at[ch0], sslot.at[nb], isem).wait()
            pltpu.make_async_copy(dstp.at[ch0], dslot.at[nb], isem).wait()
            pltpu.async_copy(z.at[sslot.at[nb]], rows.at[nb], gsem)

        pltpu.make_async_copy(z.at[pl.ds(0, CHUNK)], rows.at[b], gsem).wait()
        pltpu.sync_copy(rows.at[b], acc2.at[dslot.at[b]], add=True)

        @pl.when(i < _CHUNKS_PER_WORKER - 2)
        def _():
            pltpu.async_copy(srcp.at[ch0 + i + 2], sslot.at[b], isem)
            pltpu.async_copy(dstp.at[ch0 + i + 2], dslot.at[b], isem)

        return carry

    lax.fori_loop(0, _CHUNKS_PER_WORKER, chunk_body, 0)
    plsc.subcore_barrier()
    pltpu.sync_copy(acc2.at[pl.ds(r0, ROWS_PER_TILE)],
                    out2.at[pl.ds(c * NP + r0, ROWS_PER_TILE)])


_sc2 = pl.kernel(
    _sc2_body,
    out_type=jax.ShapeDtypeStruct((2 * NP, D_OUT), jnp.float32),
    mesh=_MESH,
    scratch_types=[
        pltpu.VMEM_SHARED((NP, D_OUT), jnp.float32),
        pltpu.VMEM((2, CHUNK), jnp.int32),
        pltpu.VMEM((2, CHUNK), jnp.int32),
        pltpu.VMEM((2, CHUNK, D_OUT), jnp.float32),
        pltpu.SemaphoreType.DMA,
        pltpu.SemaphoreType.DMA,
    ],
    compiler_params=pltpu.CompilerParams(use_tc_tiling_on_sc=False),
)


BM = 512
_PREC = lax.Precision.HIGHEST


def _tc1_body(x_r, a0_r, a1_r, p0_r, p1_r, wl1a_r, wl1b_r, wr1_r, we_r,
              bl1_r, be_r, wl2_r, h_r, z_r):
    p = p0_r[:] + p1_r[:]
    deg = p[:, 16:17]
    inv = 1.0 / jnp.maximum(deg, 1.0)
    acc = jnp.dot(a0_r[:] * inv, wl1a_r[:], precision=_PREC)
    acc = acc + jnp.dot(a1_r[:] * inv, wl1b_r[:], precision=_PREC)
    acc = acc + jnp.dot(x_r[:], wr1_r[:], precision=_PREC)
    acc = acc + jnp.dot(p[:, :D_E], we_r[:], precision=_PREC)
    acc = acc + bl1_r[:] + deg * be_r[:]
    h = jnp.maximum(acc, 0.0)
    h_r[:] = h
    z_r[:] = jnp.dot(h, wl2_r[:], precision=_PREC)


def _tc1(xp, a0, a1, p0, p1, wl1a, wl1b, wr1, we, bl1, be, wl2):
    grid = (NP // BM,)
    return pl.pallas_call(
        _tc1_body,
        grid=grid,
        in_specs=[
            pl.BlockSpec((BM, D_IN), lambda i: (i, 0)),
            pl.BlockSpec((BM, 128), lambda i: (i, 0)),
            pl.BlockSpec((BM, 128), lambda i: (i, 0)),
            pl.BlockSpec((BM, D_P), lambda i: (i, 0)),
            pl.BlockSpec((BM, D_P), lambda i: (i, 0)),
            pl.BlockSpec((128, D_H), lambda i: (0, 0)),
            pl.BlockSpec((128, D_H), lambda i: (0, 0)),
            pl.BlockSpec((D_IN, D_H), lambda i: (0, 0)),
            pl.BlockSpec((D_E, D_H), lambda i: (0, 0)),
            pl.BlockSpec((1, D_H), lambda i: (0, 0)),
            pl.BlockSpec((1, D_H), lambda i: (0, 0)),
            pl.BlockSpec((D_H, D_OUT), lambda i: (0, 0)),
        ],
        out_specs=[
            pl.BlockSpec((BM, D_H), lambda i: (i, 0)),
            pl.BlockSpec((BM, D_OUT), lambda i: (i, 0)),
        ],
        out_shape=[
            jax.ShapeDtypeStruct((NP, D_H), jnp.float32),
            jax.ShapeDtypeStruct((NP, D_OUT), jnp.float32),
        ],
    )(xp, a0, a1, p0, p1, wl1a, wl1b, wr1, we, bl1, be, wl2)


def _tc2_body(h_r, b0_r, b1_r, p0_r, p1_r, wr2_r, bl2_r, o_r):
    p = p0_r[:] + p1_r[:]
    inv = 1.0 / jnp.maximum(p[:, 16:17], 1.0)
    o = (b0_r[:] + b1_r[:]) * inv
    o = o + jnp.dot(h_r[:], wr2_r[:], precision=_PREC)
    o_r[:] = o + bl2_r[:]


def _tc2(h, b0, b1, p0, p1, wr2, bl2):
    grid = (NP // BM,)
    return pl.pallas_call(
        _tc2_body,
        grid=grid,
        in_specs=[
            pl.BlockSpec((BM, D_H), lambda i: (i, 0)),
            pl.BlockSpec((BM, D_OUT), lambda i: (i, 0)),
            pl.BlockSpec((BM, D_OUT), lambda i: (i, 0)),
            pl.BlockSpec((BM, D_P), lambda i: (i, 0)),
            pl.BlockSpec((BM, D_P), lambda i: (i, 0)),
            pl.BlockSpec((D_H, D_OUT), lambda i: (0, 0)),
            pl.BlockSpec((1, D_OUT), lambda i: (0, 0)),
        ],
        out_specs=pl.BlockSpec((BM, D_OUT), lambda i: (i, 0)),
        out_shape=jax.ShapeDtypeStruct((NP, D_OUT), jnp.float32),
    )(h, b0, b1, p0, p1, wr2, bl2)


@jax.jit
def kernel(x, edge_index, edge_attr, Wl1, bl1, Wr1, We, be, Wl2, bl2, Wr2):
    src = edge_index[0]
    dst = edge_index[1]

    xp = jnp.pad(x, ((0, NP - N_NODES), (0, 0)))
    xs = jnp.concatenate([xp[:, :128], xp[:, 128:]], axis=0)
    srcp = jnp.pad(src, (0, EP - N_EDGES)).reshape(EP // CHUNK, CHUNK)
    srcp2 = jnp.concatenate([srcp, srcp + NP], axis=0)
    dstp = jnp.pad(dst, (0, EP - N_EDGES),
                   constant_values=N_NODES).reshape(EP // CHUNK, CHUNK)
    pay = jnp.concatenate(
        [edge_attr, jnp.ones((N_EDGES, 1), jnp.float32),
         jnp.zeros((N_EDGES, D_P - D_E - 1), jnp.float32)], axis=1)
    pay = jnp.pad(pay, ((0, EP - N_EDGES), (0, 0))).astype(jnp.bfloat16)
    z128 = jnp.zeros((NP, 128), jnp.float32)
    z32 = jnp.zeros((NP, D_P), jnp.bfloat16)
    z64 = jnp.zeros((NP, D_OUT), jnp.float32)

    aggx, aggp = _sc1(xs, srcp2, dstp, pay, z128, z32)
    a0, a1 = aggx[:NP], aggx[NP:]
    p0 = aggp[:NP].astype(jnp.float32)
    p1 = aggp[NP:].astype(jnp.float32)

    h, z = _tc1(xp, a0, a1, p0, p1,
                Wl1[:128], Wl1[128:], Wr1, We,
                bl1.reshape(1, D_H), be.reshape(1, D_H), Wl2)

    agg2 = _sc2(z, srcp, dstp, z64)
    out = _tc2(h, agg2[:NP], agg2[NP:], p0, p1, Wr2, bl2.reshape(1, D_OUT))
    return out[:N_NODES]


# speedup vs baseline: 1.1363x; 1.0103x over previous
import jax
import jax.numpy as jnp
from jax import lax
from jax.experimental import pallas as pl
from jax.experimental.pallas import tpu as pltpu
from jax.experimental.pallas import tpu_sc as plsc

N_NODES = 10000
N_EDGES = 160000
D_IN = 256
D_H = 512
D_OUT = 64
D_E = 16
D_P = 32

NC = 2
NS = 16
CHUNK = 128
NP = 10240
EP = 163840
CHUNKS_PER_TILE = EP // NS // CHUNK
ROWS_PER_TILE = NP // NS

_MESH = plsc.VectorSubcoreMesh(
    core_axis_name="c", subcore_axis_name="s", num_cores=NC, num_subcores=NS)


_CHUNKS_PER_WORKER = EP // CHUNK // (NC * NS)
_ZB = ROWS_PER_TILE // CHUNK


def _sc0_body(pay, dstp, outp, accp, dslot, pv, isem):
    c = lax.axis_index("c")
    s = lax.axis_index("s")
    r0 = s * ROWS_PER_TILE

    zv = jnp.zeros((32,), jnp.bfloat16)

    def zrow(j, carry):
        pv[0, j] = zv
        return carry

    lax.fori_loop(0, CHUNK, zrow, 0)
    for k in range(_ZB):
        pltpu.sync_copy(pv.at[0], accp.at[pl.ds(r0 + k * CHUNK, CHUNK)])
    plsc.subcore_barrier()

    wid = s * NC + c
    ch0 = wid * _CHUNKS_PER_WORKER
    pltpu.sync_copy(pay.at[pl.ds(ch0 * CHUNK, CHUNK)], pv.at[0])
    pltpu.sync_copy(dstp.at[ch0], dslot.at[0])
    pltpu.async_copy(pay.at[pl.ds((ch0 + 1) * CHUNK, CHUNK)], pv.at[1], isem)
    pltpu.async_copy(dstp.at[ch0 + 1], dslot.at[1], isem)

    def chunk_body(i, carry):
        b = lax.rem(i, 2)

        @pl.when(i >= 1)
        def _():
            pltpu.make_async_copy(pay.at[pl.ds(0, CHUNK)], pv.at[b], isem).wait()
            pltpu.make_async_copy(dstp.at[ch0], dslot.at[b], isem).wait()

        pltpu.sync_copy(pv.at[b], accp.at[dslot.at[b]], add=True)

        @pl.when(i + 2 < _CHUNKS_PER_WORKER)
        def _():
            pltpu.async_copy(pay.at[pl.ds((ch0 + i + 2) * CHUNK, CHUNK)],
                             pv.at[b], isem)
            pltpu.async_copy(dstp.at[ch0 + i + 2], dslot.at[b], isem)

        return carry

    lax.fori_loop(0, _CHUNKS_PER_WORKER, chunk_body, 0)
    plsc.subcore_barrier()
    pltpu.sync_copy(accp.at[pl.ds(r0, ROWS_PER_TILE)],
                    outp.at[pl.ds(c * NP + r0, ROWS_PER_TILE)])


_sc0 = pl.kernel(
    _sc0_body,
    out_type=jax.ShapeDtypeStruct((2 * NP, D_P), jnp.bfloat16),
    mesh=_MESH,
    scratch_types=[
        pltpu.VMEM_SHARED((NP, D_P), jnp.bfloat16),
        pltpu.VMEM((2, CHUNK), jnp.int32),
        pltpu.VMEM((2, CHUNK, D_P), jnp.bfloat16),
        pltpu.SemaphoreType.DMA,
    ],
    compiler_params=pltpu.CompilerParams(use_tc_tiling_on_sc=False),
)


def _sc1_body(x, srcp, dstp, outx, accx, sslot, dslot, rows, gsem, isem):
    c = lax.axis_index("c")
    s = lax.axis_index("s")
    r0 = s * ROWS_PER_TILE
    coff = c * 128

    zv = jnp.zeros((16,), jnp.float32)

    def zrow(j, carry):
        for k in range(8):
            rows[0, j, pl.ds(16 * k, 16)] = zv
        return carry

    lax.fori_loop(0, CHUNK, zrow, 0)
    for k in range(_ZB):
        pltpu.sync_copy(rows.at[0], accx.at[pl.ds(r0 + k * CHUNK, CHUNK)])
    plsc.subcore_barrier()

    row0 = s * CHUNKS_PER_TILE

    pltpu.sync_copy(srcp.at[row0], sslot.at[0])
    pltpu.sync_copy(dstp.at[row0], dslot.at[0])
    pltpu.async_copy(srcp.at[row0 + 1], sslot.at[1], isem)
    pltpu.async_copy(dstp.at[row0 + 1], dslot.at[1], isem)
    pltpu.async_copy(x.at[sslot.at[0], pl.ds(coff, 128)], rows.at[0], gsem)

    def chunk_body(i, carry):
        b = lax.rem(i, 2)
        nb = 1 - b

        @pl.when(i < CHUNKS_PER_TILE - 1)
        def _():
            pltpu.make_async_copy(srcp.at[row0], sslot.at[nb], isem).wait()
            pltpu.make_async_copy(dstp.at[row0], dslot.at[nb], isem).wait()
            pltpu.async_copy(x.at[sslot.at[nb], pl.ds(coff, 128)],
                             rows.at[nb], gsem)

        pltpu.make_async_copy(x.at[pl.ds(0, CHUNK), pl.ds(0, 128)],
                              rows.at[b], gsem).wait()
        pltpu.sync_copy(rows.at[b], accx.at[dslot.at[b]], add=True)

        @pl.when(i < CHUNKS_PER_TILE - 2)
        def _():
            pltpu.async_copy(srcp.at[row0 + i + 2], sslot.at[b], isem)
            pltpu.async_copy(dstp.at[row0 + i + 2], dslot.at[b], isem)

        return carry

    lax.fori_loop(0, CHUNKS_PER_TILE, chunk_body, 0)
    plsc.subcore_barrier()
    pltpu.sync_copy(accx.at[pl.ds(r0, ROWS_PER_TILE)],
                    outx.at[pl.ds(c * NP + r0, ROWS_PER_TILE)])


_sc1 = pl.kernel(
    _sc1_body,
    out_type=jax.ShapeDtypeStruct((2 * NP, 128), jnp.float32),
    mesh=_MESH,
    scratch_types=[
        pltpu.VMEM_SHARED((NP, 128), jnp.float32),
        pltpu.VMEM((2, CHUNK), jnp.int32),
        pltpu.VMEM((2, CHUNK), jnp.int32),
        pltpu.VMEM((2, CHUNK, 128), jnp.float32),
        pltpu.SemaphoreType.DMA,
        pltpu.SemaphoreType.DMA,
    ],
)


def _sc2_body(z, srcp, dstp, out2, acc2, sslot, dslot, rows, gsem, isem):
    c = lax.axis_index("c")
    s = lax.axis_index("s")
    r0 = s * ROWS_PER_TILE

    zv = jnp.zeros((16,), jnp.float32)

    def zrow(j, carry):
        for k in range(D_OUT // 16):
            rows[0, j, pl.ds(16 * k, 16)] = zv
        return carry

    lax.fori_loop(0, CHUNK, zrow, 0)
    for k in range(_ZB):
        pltpu.sync_copy(rows.at[0], acc2.at[pl.ds(r0 + k * CHUNK, CHUNK)])
    plsc.subcore_barrier()

    wid = s * NC + c
    ch0 = wid * _CHUNKS_PER_WORKER

    pltpu.sync_copy(srcp.at[ch0], sslot.at[0])
    pltpu.sync_copy(dstp.at[ch0], dslot.at[0])
    pltpu.async_copy(srcp.at[ch0 + 1], sslot.at[1], isem)
    pltpu.async_copy(dstp.at[ch0 + 1], dslot.at[1], isem)
    pltpu.async_copy(z.at[sslot.at[0]], rows.at[0], gsem)

    def chunk_body(i, carry):
        b = lax.rem(i, 2)
        nb = 1 - b

        @pl.when(i < _CHUNKS_PER_WORKER - 1)
        def _():
            pltpu.make_async_copy(srcp.at[ch0], sslot.at[nb], isem).wait()
            pltpu.make_async_copy(dstp.at[ch0], dslot.at[nb], isem).wait()
            pltpu.async_copy(z.at[sslot.at[nb]], rows.at[nb], gsem)

        pltpu.make_async_copy(z.at[pl.ds(0, CHUNK)], rows.at[b], gsem).wait()
        pltpu.sync_copy(rows.at[b], acc2.at[dslot.at[b]], add=True)

        @pl.when(i < _CHUNKS_PER_WORKER - 2)
        def _():
            pltpu.async_copy(srcp.at[ch0 + i + 2], sslot.at[b], isem)
            pltpu.async_copy(dstp.at[ch0 + i + 2], dslot.at[b], isem)

        return carry

    lax.fori_loop(0, _CHUNKS_PER_WORKER, chunk_body, 0)
    plsc.subcore_barrier()
    pltpu.sync_copy(acc2.at[pl.ds(r0, ROWS_PER_TILE)],
                    out2.at[pl.ds(c * NP + r0, ROWS_PER_TILE)])


_sc2 = pl.kernel(
    _sc2_body,
    out_type=jax.ShapeDtypeStruct((2 * NP, D_OUT), jnp.float32),
    mesh=_MESH,
    scratch_types=[
        pltpu.VMEM_SHARED((NP, D_OUT), jnp.float32),
        pltpu.VMEM((2, CHUNK), jnp.int32),
        pltpu.VMEM((2, CHUNK), jnp.int32),
        pltpu.VMEM((2, CHUNK, D_OUT), jnp.float32),
        pltpu.SemaphoreType.DMA,
        pltpu.SemaphoreType.DMA,
    ],
    compiler_params=pltpu.CompilerParams(use_tc_tiling_on_sc=False),
)


BM = 512
_PREC = lax.Precision.HIGHEST


def _tc1_body(x_r, a0_r, a1_r, p0_r, p1_r, wl1a_r, wl1b_r, wr1_r, we_r,
              bl1_r, be_r, wl2_r, h_r, z_r):
    p = p0_r[:].astype(jnp.float32) + p1_r[:].astype(jnp.float32)
    deg = p[:, 16:17]
    inv = 1.0 / jnp.maximum(deg, 1.0)
    acc = jnp.dot(a0_r[:] * inv, wl1a_r[:], precision=_PREC)
    acc = acc + jnp.dot(a1_r[:] * inv, wl1b_r[:], precision=_PREC)
    acc = acc + jnp.dot(x_r[:], wr1_r[:], precision=_PREC)
    acc = acc + jnp.dot(p[:, :D_E], we_r[:], precision=_PREC)
    acc = acc + bl1_r[:] + deg * be_r[:]
    h = jnp.maximum(acc, 0.0)
    h_r[:] = h
    z_r[:] = jnp.dot(h, wl2_r[:], precision=_PREC)


def _tc1(xp, a0, a1, p0, p1, wl1a, wl1b, wr1, we, bl1, be, wl2):
    grid = (NP // BM,)
    return pl.pallas_call(
        _tc1_body,
        grid=grid,
        in_specs=[
            pl.BlockSpec((BM, D_IN), lambda i: (i, 0)),
            pl.BlockSpec((BM, 128), lambda i: (i, 0)),
            pl.BlockSpec((BM, 128), lambda i: (i, 0)),
            pl.BlockSpec((BM, D_P), lambda i: (i, 0)),
            pl.BlockSpec((BM, D_P), lambda i: (i, 0)),
            pl.BlockSpec((128, D_H), lambda i: (0, 0)),
            pl.BlockSpec((128, D_H), lambda i: (0, 0)),
            pl.BlockSpec((D_IN, D_H), lambda i: (0, 0)),
            pl.BlockSpec((D_E, D_H), lambda i: (0, 0)),
            pl.BlockSpec((1, D_H), lambda i: (0, 0)),
            pl.BlockSpec((1, D_H), lambda i: (0, 0)),
            pl.BlockSpec((D_H, D_OUT), lambda i: (0, 0)),
        ],
        out_specs=[
            pl.BlockSpec((BM, D_H), lambda i: (i, 0)),
            pl.BlockSpec((BM, D_OUT), lambda i: (i, 0)),
        ],
        out_shape=[
            jax.ShapeDtypeStruct((NP, D_H), jnp.float32),
            jax.ShapeDtypeStruct((NP, D_OUT), jnp.float32),
        ],
    )(xp, a0, a1, p0, p1, wl1a, wl1b, wr1, we, bl1, be, wl2)


def _tc2_body(h_r, b0_r, b1_r, p0_r, p1_r, wr2_r, bl2_r, o_r):
    p = p0_r[:].astype(jnp.float32) + p1_r[:].astype(jnp.float32)
    inv = 1.0 / jnp.maximum(p[:, 16:17], 1.0)
    o = (b0_r[:] + b1_r[:]) * inv
    o = o + jnp.dot(h_r[:], wr2_r[:], precision=_PREC)
    o_r[:] = o + bl2_r[:]


def _tc2(h, b0, b1, p0, p1, wr2, bl2):
    grid = (NP // BM,)
    return pl.pallas_call(
        _tc2_body,
        grid=grid,
        in_specs=[
            pl.BlockSpec((BM, D_H), lambda i: (i, 0)),
            pl.BlockSpec((BM, D_OUT), lambda i: (i, 0)),
            pl.BlockSpec((BM, D_OUT), lambda i: (i, 0)),
            pl.BlockSpec((BM, D_P), lambda i: (i, 0)),
            pl.BlockSpec((BM, D_P), lambda i: (i, 0)),
            pl.BlockSpec((D_H, D_OUT), lambda i: (0, 0)),
            pl.BlockSpec((1, D_OUT), lambda i: (0, 0)),
        ],
        out_specs=pl.BlockSpec((BM, D_OUT), lambda i: (i, 0)),
        out_shape=jax.ShapeDtypeStruct((NP, D_OUT), jnp.float32),
    )(h, b0, b1, p0, p1, wr2, bl2)


@jax.jit
def kernel(x, edge_index, edge_attr, Wl1, bl1, Wr1, We, be, Wl2, bl2, Wr2):
    src = edge_index[0]
    dst = edge_index[1]

    xp = jnp.pad(x, ((0, NP - N_NODES), (0, 0)))
    srcp = jnp.pad(src, (0, EP - N_EDGES)).reshape(EP // CHUNK, CHUNK)
    dstp = jnp.pad(dst, (0, EP - N_EDGES),
                   constant_values=N_NODES).reshape(EP // CHUNK, CHUNK)
    pay = jnp.concatenate(
        [edge_attr, jnp.ones((N_EDGES, 1), jnp.float32),
         jnp.zeros((N_EDGES, D_P - D_E - 1), jnp.float32)], axis=1)
    pay = jnp.pad(pay, ((0, EP - N_EDGES), (0, 0))).astype(jnp.bfloat16)

    aggp = _sc0(pay, dstp)
    aggx = _sc1(x, srcp, dstp)
    a0, a1 = aggx[:NP], aggx[NP:]
    p0, p1 = aggp[:NP], aggp[NP:]

    h, z = _tc1(xp, a0, a1, p0, p1,
                Wl1[:128], Wl1[128:], Wr1, We,
                bl1.reshape(1, D_H), be.reshape(1, D_H), Wl2)

    agg2 = _sc2(z, srcp, dstp)
    out = _tc2(h, agg2[:NP], agg2[NP:], p0, p1, Wr2, bl2.reshape(1, D_OUT))
    return out[:N_NODES]


# speedup vs baseline: 1.2919x; 1.1369x over previous
import jax
import jax.numpy as jnp
from jax import lax
from jax.experimental import pallas as pl
from jax.experimental.pallas import tpu as pltpu
from jax.experimental.pallas import tpu_sc as plsc

N_NODES = 10000
N_EDGES = 160000
D_IN = 256
D_H = 512
D_OUT = 64
D_E = 16
D_P = 32

NC = 2
NS = 16
CHUNK = 128
NP = 10240
EP = 163840
CHUNKS_PER_TILE = EP // NS // CHUNK
ROWS_PER_TILE = NP // NS

_MESH = plsc.VectorSubcoreMesh(
    core_axis_name="c", subcore_axis_name="s", num_cores=NC, num_subcores=NS)


_CHUNKS_PER_WORKER = EP // CHUNK // (NC * NS)
_ZB = ROWS_PER_TILE // CHUNK


def _sc0_body(pay, dstp, outp, accp, dslot, pv, isem):
    c = lax.axis_index("c")
    s = lax.axis_index("s")
    r0 = s * ROWS_PER_TILE

    zv = jnp.zeros((32,), jnp.bfloat16)

    def zrow(j, carry):
        pv[0, j] = zv
        return carry

    lax.fori_loop(0, CHUNK, zrow, 0)
    for k in range(_ZB):
        pltpu.sync_copy(pv.at[0], accp.at[pl.ds(r0 + k * CHUNK, CHUNK)])
    plsc.subcore_barrier()

    wid = s * NC + c
    ch0 = wid * _CHUNKS_PER_WORKER
    pltpu.sync_copy(pay.at[pl.ds(ch0 * CHUNK, CHUNK)], pv.at[0])
    pltpu.sync_copy(dstp.at[ch0], dslot.at[0])
    pltpu.async_copy(pay.at[pl.ds((ch0 + 1) * CHUNK, CHUNK)], pv.at[1], isem)
    pltpu.async_copy(dstp.at[ch0 + 1], dslot.at[1], isem)

    def chunk_body(i, carry):
        b = lax.rem(i, 2)

        @pl.when(i >= 1)
        def _():
            pltpu.make_async_copy(pay.at[pl.ds(0, CHUNK)], pv.at[b], isem).wait()
            pltpu.make_async_copy(dstp.at[ch0], dslot.at[b], isem).wait()

        pltpu.sync_copy(pv.at[b], accp.at[dslot.at[b]], add=True)

        @pl.when(i + 2 < _CHUNKS_PER_WORKER)
        def _():
            pltpu.async_copy(pay.at[pl.ds((ch0 + i + 2) * CHUNK, CHUNK)],
                             pv.at[b], isem)
            pltpu.async_copy(dstp.at[ch0 + i + 2], dslot.at[b], isem)

        return carry

    lax.fori_loop(0, _CHUNKS_PER_WORKER, chunk_body, 0)
    plsc.subcore_barrier()
    pltpu.sync_copy(accp.at[pl.ds(r0, ROWS_PER_TILE)],
                    outp.at[pl.ds(c * NP + r0, ROWS_PER_TILE)])


_sc0 = pl.kernel(
    _sc0_body,
    out_type=jax.ShapeDtypeStruct((2 * NP, D_P), jnp.bfloat16),
    mesh=_MESH,
    scratch_types=[
        pltpu.VMEM_SHARED((NP, D_P), jnp.bfloat16),
        pltpu.VMEM((2, CHUNK), jnp.int32),
        pltpu.VMEM((2, CHUNK, D_P), jnp.bfloat16),
        pltpu.SemaphoreType.DMA,
    ],
    compiler_params=pltpu.CompilerParams(use_tc_tiling_on_sc=False),
)


_IB = 8


def _sc1_body(x, srcp, dstp, outx, accx, sslot, dslot, rows, gsem):
    c = lax.axis_index("c")
    s = lax.axis_index("s")
    r0 = s * ROWS_PER_TILE
    coff = c * 128

    zv = jnp.zeros((16,), jnp.float32)

    def zrow(j, carry):
        for k in range(8):
            rows[0, j, pl.ds(16 * k, 16)] = zv
        return carry

    lax.fori_loop(0, CHUNK, zrow, 0)
    for k in range(_ZB):
        pltpu.sync_copy(rows.at[0], accx.at[pl.ds(r0 + k * CHUNK, CHUNK)])
    plsc.subcore_barrier()

    row0 = s * CHUNKS_PER_TILE

    pltpu.sync_copy(srcp.at[pl.ds(row0, _IB)], sslot.at[0])
    pltpu.sync_copy(dstp.at[pl.ds(row0, _IB)], dslot.at[0])
    pltpu.async_copy(x.at[sslot.at[0, 0], pl.ds(coff, 128)], rows.at[0], gsem)

    def chunk_body(i, carry):
        b = lax.rem(i, 2)
        nb = 1 - b
        ib = lax.rem(i // _IB, 2)

        @pl.when((lax.rem(i, _IB) == _IB - 1) & (i < CHUNKS_PER_TILE - 1))
        def _():
            nxt = pl.multiple_of(row0 + i + 1, _IB)
            pltpu.sync_copy(srcp.at[pl.ds(nxt, _IB)], sslot.at[1 - ib])
            pltpu.sync_copy(dstp.at[pl.ds(nxt, _IB)], dslot.at[1 - ib])

        @pl.when(i < CHUNKS_PER_TILE - 1)
        def _():
            j1 = lax.rem(i + 1, _IB)
            ib1 = lax.rem((i + 1) // _IB, 2)
            pltpu.async_copy(x.at[sslot.at[ib1, j1], pl.ds(coff, 128)],
                             rows.at[nb], gsem)

        pltpu.make_async_copy(x.at[pl.ds(0, CHUNK), pl.ds(0, 128)],
                              rows.at[b], gsem).wait()
        pltpu.sync_copy(rows.at[b], accx.at[dslot.at[ib, lax.rem(i, _IB)]],
                        add=True)
        return carry

    lax.fori_loop(0, CHUNKS_PER_TILE, chunk_body, 0)
    plsc.subcore_barrier()
    pltpu.sync_copy(accx.at[pl.ds(r0, ROWS_PER_TILE)],
                    outx.at[pl.ds(c * NP + r0, ROWS_PER_TILE)])


_sc1 = pl.kernel(
    _sc1_body,
    out_type=jax.ShapeDtypeStruct((2 * NP, 128), jnp.float32),
    mesh=_MESH,
    scratch_types=[
        pltpu.VMEM_SHARED((NP, 128), jnp.float32),
        pltpu.VMEM((2, _IB, CHUNK), jnp.int32),
        pltpu.VMEM((2, _IB, CHUNK), jnp.int32),
        pltpu.VMEM((2, CHUNK, 128), jnp.float32),
        pltpu.SemaphoreType.DMA,
    ],
)


def _sc2_body(z, srcp, dstp, out2, acc2, sslot, dslot, rows, gsem):
    c = lax.axis_index("c")
    s = lax.axis_index("s")
    r0 = s * ROWS_PER_TILE

    zv = jnp.zeros((16,), jnp.float32)

    def zrow(j, carry):
        for k in range(D_OUT // 16):
            rows[0, j, pl.ds(16 * k, 16)] = zv
        return carry

    lax.fori_loop(0, CHUNK, zrow, 0)
    for k in range(_ZB):
        pltpu.sync_copy(rows.at[0], acc2.at[pl.ds(r0 + k * CHUNK, CHUNK)])
    plsc.subcore_barrier()

    wid = s * NC + c
    ch0 = wid * _CHUNKS_PER_WORKER

    pltpu.sync_copy(srcp.at[pl.ds(ch0, _IB)], sslot.at[0])
    pltpu.sync_copy(dstp.at[pl.ds(ch0, _IB)], dslot.at[0])
    pltpu.async_copy(z.at[sslot.at[0, 0]], rows.at[0], gsem)

    def chunk_body(i, carry):
        b = lax.rem(i, 2)
        nb = 1 - b
        ib = lax.rem(i // _IB, 2)

        @pl.when((lax.rem(i, _IB) == _IB - 1) & (i < _CHUNKS_PER_WORKER - 1))
        def _():
            nxt = pl.multiple_of(ch0 + i + 1, _IB)
            pltpu.sync_copy(srcp.at[pl.ds(nxt, _IB)], sslot.at[1 - ib])
            pltpu.sync_copy(dstp.at[pl.ds(nxt, _IB)], dslot.at[1 - ib])

        @pl.when(i < _CHUNKS_PER_WORKER - 1)
        def _():
            j1 = lax.rem(i + 1, _IB)
            ib1 = lax.rem((i + 1) // _IB, 2)
            pltpu.async_copy(z.at[sslot.at[ib1, j1]], rows.at[nb], gsem)

        pltpu.make_async_copy(z.at[pl.ds(0, CHUNK)], rows.at[b], gsem).wait()
        pltpu.sync_copy(rows.at[b], acc2.at[dslot.at[ib, lax.rem(i, _IB)]],
                        add=True)
        return carry

    lax.fori_loop(0, _CHUNKS_PER_WORKER, chunk_body, 0)
    plsc.subcore_barrier()
    pltpu.sync_copy(acc2.at[pl.ds(r0, ROWS_PER_TILE)],
                    out2.at[pl.ds(c * NP + r0, ROWS_PER_TILE)])


_sc2 = pl.kernel(
    _sc2_body,
    out_type=jax.ShapeDtypeStruct((2 * NP, D_OUT), jnp.float32),
    mesh=_MESH,
    scratch_types=[
        pltpu.VMEM_SHARED((NP, D_OUT), jnp.float32),
        pltpu.VMEM((2, _IB, CHUNK), jnp.int32),
        pltpu.VMEM((2, _IB, CHUNK), jnp.int32),
        pltpu.VMEM((2, CHUNK, D_OUT), jnp.float32),
        pltpu.SemaphoreType.DMA,
    ],
    compiler_params=pltpu.CompilerParams(use_tc_tiling_on_sc=False),
)


BM = 512
_PREC = None


def _tc1_body(x_r, a0_r, a1_r, p0_r, p1_r, wl1a_r, wl1b_r, wr1_r, we_r,
              bl1_r, be_r, wl2_r, h_r, z_r):
    p = p0_r[:].astype(jnp.float32) + p1_r[:].astype(jnp.float32)
    deg = p[:, 16:17]
    inv = 1.0 / jnp.maximum(deg, 1.0)
    acc = jnp.dot(a0_r[:] * inv, wl1a_r[:], precision=_PREC)
    acc = acc + jnp.dot(a1_r[:] * inv, wl1b_r[:], precision=_PREC)
    acc = acc + jnp.dot(x_r[:], wr1_r[:], precision=_PREC)
    acc = acc + jnp.dot(p[:, :D_E], we_r[:], precision=_PREC)
    acc = acc + bl1_r[:] + deg * be_r[:]
    h = jnp.maximum(acc, 0.0)
    h_r[:] = h
    z_r[:] = jnp.dot(h, wl2_r[:], precision=_PREC)


def _tc1(xp, a0, a1, p0, p1, wl1a, wl1b, wr1, we, bl1, be, wl2):
    grid = (NP // BM,)
    return pl.pallas_call(
        _tc1_body,
        grid=grid,
        in_specs=[
            pl.BlockSpec((BM, D_IN), lambda i: (i, 0)),
            pl.BlockSpec((BM, 128), lambda i: (i, 0)),
            pl.BlockSpec((BM, 128), lambda i: (i, 0)),
            pl.BlockSpec((BM, D_P), lambda i: (i, 0)),
            pl.BlockSpec((BM, D_P), lambda i: (i, 0)),
            pl.BlockSpec((128, D_H), lambda i: (0, 0)),
            pl.BlockSpec((128, D_H), lambda i: (0, 0)),
            pl.BlockSpec((D_IN, D_H), lambda i: (0, 0)),
            pl.BlockSpec((D_E, D_H), lambda i: (0, 0)),
            pl.BlockSpec((1, D_H), lambda i: (0, 0)),
            pl.BlockSpec((1, D_H), lambda i: (0, 0)),
            pl.BlockSpec((D_H, D_OUT), lambda i: (0, 0)),
        ],
        out_specs=[
            pl.BlockSpec((BM, D_H), lambda i: (i, 0)),
            pl.BlockSpec((BM, D_OUT), lambda i: (i, 0)),
        ],
        out_shape=[
            jax.ShapeDtypeStruct((NP, D_H), jnp.float32),
            jax.ShapeDtypeStruct((NP, D_OUT), jnp.float32),
        ],
    )(xp, a0, a1, p0, p1, wl1a, wl1b, wr1, we, bl1, be, wl2)


def _tc2_body(h_r, b0_r, b1_r, p0_r, p1_r, wr2_r, bl2_r, o_r):
    p = p0_r[:].astype(jnp.float32) + p1_r[:].astype(jnp.float32)
    inv = 1.0 / jnp.maximum(p[:, 16:17], 1.0)
    o = (b0_r[:] + b1_r[:]) * inv
    o = o + jnp.dot(h_r[:], wr2_r[:], precision=_PREC)
    o_r[:] = o + bl2_r[:]


def _tc2(h, b0, b1, p0, p1, wr2, bl2):
    grid = (NP // BM,)
    return pl.pallas_call(
        _tc2_body,
        grid=grid,
        in_specs=[
            pl.BlockSpec((BM, D_H), lambda i: (i, 0)),
            pl.BlockSpec((BM, D_OUT), lambda i: (i, 0)),
            pl.BlockSpec((BM, D_OUT), lambda i: (i, 0)),
            pl.BlockSpec((BM, D_P), lambda i: (i, 0)),
            pl.BlockSpec((BM, D_P), lambda i: (i, 0)),
            pl.BlockSpec((D_H, D_OUT), lambda i: (0, 0)),
            pl.BlockSpec((1, D_OUT), lambda i: (0, 0)),
        ],
        out_specs=pl.BlockSpec((BM, D_OUT), lambda i: (i, 0)),
        out_shape=jax.ShapeDtypeStruct((NP, D_OUT), jnp.float32),
    )(h, b0, b1, p0, p1, wr2, bl2)


@jax.jit
def kernel(x, edge_index, edge_attr, Wl1, bl1, Wr1, We, be, Wl2, bl2, Wr2):
    src = edge_index[0]
    dst = edge_index[1]

    xp = jnp.pad(x, ((0, NP - N_NODES), (0, 0)))
    srcp = jnp.pad(src, (0, EP - N_EDGES)).reshape(EP // CHUNK, CHUNK)
    dstp = jnp.pad(dst, (0, EP - N_EDGES),
                   constant_values=N_NODES).reshape(EP // CHUNK, CHUNK)
    pay = jnp.concatenate(
        [edge_attr, jnp.ones((N_EDGES, 1), jnp.float32),
         jnp.zeros((N_EDGES, D_P - D_E - 1), jnp.float32)], axis=1)
    pay = jnp.pad(pay, ((0, EP - N_EDGES), (0, 0))).astype(jnp.bfloat16)

    aggp = _sc0(pay, dstp)
    aggx = _sc1(x, srcp, dstp)
    a0, a1 = aggx[:NP], aggx[NP:]
    p0, p1 = aggp[:NP], aggp[NP:]

    h, z = _tc1(xp, a0, a1, p0, p1,
                Wl1[:128], Wl1[128:], Wr1, We,
                bl1.reshape(1, D_H), be.reshape(1, D_H), Wl2)

    agg2 = _sc2(z, srcp, dstp)
    out = _tc2(h, agg2[:NP], agg2[NP:], p0, p1, Wr2, bl2.reshape(1, D_OUT))
    return out[:N_NODES]


# speedup vs baseline: 1.2942x; 1.0018x over previous
import jax
import jax.numpy as jnp
from jax import lax
from jax.experimental import pallas as pl
from jax.experimental.pallas import tpu as pltpu
from jax.experimental.pallas import tpu_sc as plsc

N_NODES = 10000
N_EDGES = 160000
D_IN = 256
D_H = 512
D_OUT = 64
D_E = 16
D_P = 32

NC = 2
NS = 16
CHUNK = 128
NP = 10240
EP = 163840
CHUNKS_PER_TILE = EP // NS // CHUNK
ROWS_PER_TILE = NP // NS

_MESH = plsc.VectorSubcoreMesh(
    core_axis_name="c", subcore_axis_name="s", num_cores=NC, num_subcores=NS)


_CHUNKS_PER_WORKER = EP // CHUNK // (NC * NS)
_ZB = ROWS_PER_TILE // CHUNK


def _sc0_body(pay, dstp, outp, accp, dslot, pv, isem):
    c = lax.axis_index("c")
    s = lax.axis_index("s")
    r0 = s * ROWS_PER_TILE

    zv = jnp.zeros((32,), jnp.bfloat16)

    def zrow(j, carry):
        pv[0, j] = zv
        return carry

    lax.fori_loop(0, CHUNK, zrow, 0)
    for k in range(_ZB):
        pltpu.sync_copy(pv.at[0], accp.at[pl.ds(r0 + k * CHUNK, CHUNK)])
    plsc.subcore_barrier()

    wid = s * NC + c
    ch0 = wid * _CHUNKS_PER_WORKER
    pltpu.sync_copy(pay.at[pl.ds(ch0 * CHUNK, CHUNK)], pv.at[0])
    pltpu.sync_copy(dstp.at[ch0], dslot.at[0])
    pltpu.async_copy(pay.at[pl.ds((ch0 + 1) * CHUNK, CHUNK)], pv.at[1], isem)
    pltpu.async_copy(dstp.at[ch0 + 1], dslot.at[1], isem)

    def chunk_body(i, carry):
        b = lax.rem(i, 2)

        @pl.when(i >= 1)
        def _():
            pltpu.make_async_copy(pay.at[pl.ds(0, CHUNK)], pv.at[b], isem).wait()
            pltpu.make_async_copy(dstp.at[ch0], dslot.at[b], isem).wait()

        pltpu.sync_copy(pv.at[b], accp.at[dslot.at[b]], add=True)

        @pl.when(i + 2 < _CHUNKS_PER_WORKER)
        def _():
            pltpu.async_copy(pay.at[pl.ds((ch0 + i + 2) * CHUNK, CHUNK)],
                             pv.at[b], isem)
            pltpu.async_copy(dstp.at[ch0 + i + 2], dslot.at[b], isem)

        return carry

    lax.fori_loop(0, _CHUNKS_PER_WORKER, chunk_body, 0)
    plsc.subcore_barrier()
    pltpu.sync_copy(accp.at[pl.ds(r0, ROWS_PER_TILE)],
                    outp.at[pl.ds(c * NP + r0, ROWS_PER_TILE)])


_sc0 = pl.kernel(
    _sc0_body,
    out_type=jax.ShapeDtypeStruct((2 * NP, D_P), jnp.bfloat16),
    mesh=_MESH,
    scratch_types=[
        pltpu.VMEM_SHARED((NP, D_P), jnp.bfloat16),
        pltpu.VMEM((2, CHUNK), jnp.int32),
        pltpu.VMEM((2, CHUNK, D_P), jnp.bfloat16),
        pltpu.SemaphoreType.DMA,
    ],
    compiler_params=pltpu.CompilerParams(use_tc_tiling_on_sc=False),
)


_IB = 8


def _sc1_body(x, srcp, dstp, outx, accx, sslot, dslot, rows, gsem, ssem):
    c = lax.axis_index("c")
    s = lax.axis_index("s")
    r0 = s * ROWS_PER_TILE
    coff = c * 128

    zv = jnp.zeros((16,), jnp.float32)

    def zrow(j, carry):
        for k in range(8):
            rows[0, j, pl.ds(16 * k, 16)] = zv
        return carry

    lax.fori_loop(0, CHUNK, zrow, 0)
    for k in range(_ZB):
        pltpu.sync_copy(rows.at[0], accx.at[pl.ds(r0 + k * CHUNK, CHUNK)])
    plsc.subcore_barrier()

    row0 = s * CHUNKS_PER_TILE

    pltpu.sync_copy(srcp.at[pl.ds(row0, _IB)], sslot.at[0])
    pltpu.sync_copy(dstp.at[pl.ds(row0, _IB)], dslot.at[0])
    pltpu.async_copy(x.at[sslot.at[0, 0], pl.ds(coff, 128)], rows.at[0], gsem)

    def chunk_body(i, carry):
        b = lax.rem(i, 2)
        nb = 1 - b
        ib = lax.rem(i // _IB, 2)

        @pl.when((lax.rem(i, _IB) == _IB - 1) & (i < CHUNKS_PER_TILE - 1))
        def _():
            nxt = pl.multiple_of(row0 + i + 1, _IB)
            pltpu.sync_copy(srcp.at[pl.ds(nxt, _IB)], sslot.at[1 - ib])
            pltpu.sync_copy(dstp.at[pl.ds(nxt, _IB)], dslot.at[1 - ib])

        @pl.when(i < CHUNKS_PER_TILE - 1)
        def _():
            @pl.when(i >= 1)
            def _():
                pltpu.make_async_copy(x.at[pl.ds(0, CHUNK), pl.ds(0, 128)],
                                      rows.at[nb], ssem).wait()

            j1 = lax.rem(i + 1, _IB)
            ib1 = lax.rem((i + 1) // _IB, 2)
            pltpu.async_copy(x.at[sslot.at[ib1, j1], pl.ds(coff, 128)],
                             rows.at[nb], gsem)

        pltpu.make_async_copy(x.at[pl.ds(0, CHUNK), pl.ds(0, 128)],
                              rows.at[b], gsem).wait()
        pltpu.async_copy(rows.at[b], accx.at[dslot.at[ib, lax.rem(i, _IB)]],
                         ssem, add=True)
        return carry

    lax.fori_loop(0, CHUNKS_PER_TILE, chunk_body, 0)
    for _ in range(2):
        pltpu.make_async_copy(x.at[pl.ds(0, CHUNK), pl.ds(0, 128)],
                              rows.at[0], ssem).wait()
    plsc.subcore_barrier()
    pltpu.sync_copy(accx.at[pl.ds(r0, ROWS_PER_TILE)],
                    outx.at[pl.ds(c * NP + r0, ROWS_PER_TILE)])


_sc1 = pl.kernel(
    _sc1_body,
    out_type=jax.ShapeDtypeStruct((2 * NP, 128), jnp.float32),
    mesh=_MESH,
    scratch_types=[
        pltpu.VMEM_SHARED((NP, 128), jnp.float32),
        pltpu.VMEM((2, _IB, CHUNK), jnp.int32),
        pltpu.VMEM((2, _IB, CHUNK), jnp.int32),
        pltpu.VMEM((2, CHUNK, 128), jnp.float32),
        pltpu.SemaphoreType.DMA,
        pltpu.SemaphoreType.DMA,
    ],
)


def _sc2_body(z, srcp, dstp, out2, acc2, sslot, dslot, rows, gsem, ssem):
    c = lax.axis_index("c")
    s = lax.axis_index("s")
    r0 = s * ROWS_PER_TILE

    zv = jnp.zeros((16,), jnp.float32)

    def zrow(j, carry):
        for k in range(D_OUT // 16):
            rows[0, j, pl.ds(16 * k, 16)] = zv
        return carry

    lax.fori_loop(0, CHUNK, zrow, 0)
    for k in range(_ZB):
        pltpu.sync_copy(rows.at[0], acc2.at[pl.ds(r0 + k * CHUNK, CHUNK)])
    plsc.subcore_barrier()

    wid = s * NC + c
    ch0 = wid * _CHUNKS_PER_WORKER

    pltpu.sync_copy(srcp.at[pl.ds(ch0, _IB)], sslot.at[0])
    pltpu.sync_copy(dstp.at[pl.ds(ch0, _IB)], dslot.at[0])
    pltpu.async_copy(z.at[sslot.at[0, 0]], rows.at[0], gsem)

    def chunk_body(i, carry):
        b = lax.rem(i, 2)
        nb = 1 - b
        ib = lax.rem(i // _IB, 2)

        @pl.when((lax.rem(i, _IB) == _IB - 1) & (i < _CHUNKS_PER_WORKER - 1))
        def _():
            nxt = pl.multiple_of(ch0 + i + 1, _IB)
            pltpu.sync_copy(srcp.at[pl.ds(nxt, _IB)], sslot.at[1 - ib])
            pltpu.sync_copy(dstp.at[pl.ds(nxt, _IB)], dslot.at[1 - ib])

        @pl.when(i < _CHUNKS_PER_WORKER - 1)
        def _():
            @pl.when(i >= 1)
            def _():
                pltpu.make_async_copy(z.at[pl.ds(0, CHUNK)], rows.at[nb],
                                      ssem).wait()

            j1 = lax.rem(i + 1, _IB)
            ib1 = lax.rem((i + 1) // _IB, 2)
            pltpu.async_copy(z.at[sslot.at[ib1, j1]], rows.at[nb], gsem)

        pltpu.make_async_copy(z.at[pl.ds(0, CHUNK)], rows.at[b], gsem).wait()
        pltpu.async_copy(rows.at[b], acc2.at[dslot.at[ib, lax.rem(i, _IB)]],
                         ssem, add=True)
        return carry

    lax.fori_loop(0, _CHUNKS_PER_WORKER, chunk_body, 0)
    for _ in range(2):
        pltpu.make_async_copy(z.at[pl.ds(0, CHUNK)], rows.at[0], ssem).wait()
    plsc.subcore_barrier()
    pltpu.sync_copy(acc2.at[pl.ds(r0, ROWS_PER_TILE)],
                    out2.at[pl.ds(c * NP + r0, ROWS_PER_TILE)])


_sc2 = pl.kernel(
    _sc2_body,
    out_type=jax.ShapeDtypeStruct((2 * NP, D_OUT), jnp.float32),
    mesh=_MESH,
    scratch_types=[
        pltpu.VMEM_SHARED((NP, D_OUT), jnp.float32),
        pltpu.VMEM((2, _IB, CHUNK), jnp.int32),
        pltpu.VMEM((2, _IB, CHUNK), jnp.int32),
        pltpu.VMEM((2, CHUNK, D_OUT), jnp.float32),
        pltpu.SemaphoreType.DMA,
        pltpu.SemaphoreType.DMA,
    ],
    compiler_params=pltpu.CompilerParams(use_tc_tiling_on_sc=False),
)


BM = 512
_PREC = None


def _tc1_body(x_r, a0_r, a1_r, p0_r, p1_r, wl1a_r, wl1b_r, wr1_r, we_r,
              bl1_r, be_r, wl2_r, h_r, z_r):
    p = p0_r[:].astype(jnp.float32) + p1_r[:].astype(jnp.float32)
    deg = p[:, 16:17]
    inv = 1.0 / jnp.maximum(deg, 1.0)
    acc = jnp.dot(a0_r[:] * inv, wl1a_r[:], precision=_PREC)
    acc = acc + jnp.dot(a1_r[:] * inv, wl1b_r[:], precision=_PREC)
    acc = acc + jnp.dot(x_r[:], wr1_r[:], precision=_PREC)
    acc = acc + jnp.dot(p[:, :D_E], we_r[:], precision=_PREC)
    acc = acc + bl1_r[:] + deg * be_r[:]
    h = jnp.maximum(acc, 0.0)
    h_r[:] = h
    z_r[:] = jnp.dot(h, wl2_r[:], precision=_PREC)


def _tc1(xp, a0, a1, p0, p1, wl1a, wl1b, wr1, we, bl1, be, wl2):
    grid = (NP // BM,)
    return pl.pallas_call(
        _tc1_body,
        grid=grid,
        in_specs=[
            pl.BlockSpec((BM, D_IN), lambda i: (i, 0)),
            pl.BlockSpec((BM, 128), lambda i: (i, 0)),
            pl.BlockSpec((BM, 128), lambda i: (i, 0)),
            pl.BlockSpec((BM, D_P), lambda i: (i, 0)),
            pl.BlockSpec((BM, D_P), lambda i: (i, 0)),
            pl.BlockSpec((128, D_H), lambda i: (0, 0)),
            pl.BlockSpec((128, D_H), lambda i: (0, 0)),
            pl.BlockSpec((D_IN, D_H), lambda i: (0, 0)),
            pl.BlockSpec((D_E, D_H), lambda i: (0, 0)),
            pl.BlockSpec((1, D_H), lambda i: (0, 0)),
            pl.BlockSpec((1, D_H), lambda i: (0, 0)),
            pl.BlockSpec((D_H, D_OUT), lambda i: (0, 0)),
        ],
        out_specs=[
            pl.BlockSpec((BM, D_H), lambda i: (i, 0)),
            pl.BlockSpec((BM, D_OUT), lambda i: (i, 0)),
        ],
        out_shape=[
            jax.ShapeDtypeStruct((NP, D_H), jnp.float32),
            jax.ShapeDtypeStruct((NP, D_OUT), jnp.float32),
        ],
    )(xp, a0, a1, p0, p1, wl1a, wl1b, wr1, we, bl1, be, wl2)


def _tc2_body(h_r, b0_r, b1_r, p0_r, p1_r, wr2_r, bl2_r, o_r):
    p = p0_r[:].astype(jnp.float32) + p1_r[:].astype(jnp.float32)
    inv = 1.0 / jnp.maximum(p[:, 16:17], 1.0)
    o = (b0_r[:] + b1_r[:]) * inv
    o = o + jnp.dot(h_r[:], wr2_r[:], precision=_PREC)
    o_r[:] = o + bl2_r[:]


def _tc2(h, b0, b1, p0, p1, wr2, bl2):
    grid = (NP // BM,)
    return pl.pallas_call(
        _tc2_body,
        grid=grid,
        in_specs=[
            pl.BlockSpec((BM, D_H), lambda i: (i, 0)),
            pl.BlockSpec((BM, D_OUT), lambda i: (i, 0)),
            pl.BlockSpec((BM, D_OUT), lambda i: (i, 0)),
            pl.BlockSpec((BM, D_P), lambda i: (i, 0)),
            pl.BlockSpec((BM, D_P), lambda i: (i, 0)),
            pl.BlockSpec((D_H, D_OUT), lambda i: (0, 0)),
            pl.BlockSpec((1, D_OUT), lambda i: (0, 0)),
        ],
        out_specs=pl.BlockSpec((BM, D_OUT), lambda i: (i, 0)),
        out_shape=jax.ShapeDtypeStruct((NP, D_OUT), jnp.float32),
    )(h, b0, b1, p0, p1, wr2, bl2)


@jax.jit
def kernel(x, edge_index, edge_attr, Wl1, bl1, Wr1, We, be, Wl2, bl2, Wr2):
    src = edge_index[0]
    dst = edge_index[1]

    xp = jnp.pad(x, ((0, NP - N_NODES), (0, 0)))
    srcp = jnp.pad(src, (0, EP - N_EDGES)).reshape(EP // CHUNK, CHUNK)
    dstp = jnp.pad(dst, (0, EP - N_EDGES),
                   constant_values=N_NODES).reshape(EP // CHUNK, CHUNK)
    pay = jnp.concatenate(
        [edge_attr, jnp.ones((N_EDGES, 1), jnp.float32),
         jnp.zeros((N_EDGES, D_P - D_E - 1), jnp.float32)], axis=1)
    pay = jnp.pad(pay, ((0, EP - N_EDGES), (0, 0))).astype(jnp.bfloat16)

    aggp = _sc0(pay, dstp)
    aggx = _sc1(x, srcp, dstp)
    a0, a1 = aggx[:NP], aggx[NP:]
    p0, p1 = aggp[:NP], aggp[NP:]

    h, z = _tc1(xp, a0, a1, p0, p1,
                Wl1[:128], Wl1[128:], Wr1, We,
                bl1.reshape(1, D_H), be.reshape(1, D_H), Wl2)

    agg2 = _sc2(z, srcp, dstp)
    out = _tc2(h, agg2[:NP], agg2[NP:], p0, p1, Wr2, bl2.reshape(1, D_OUT))
    return out[:N_NODES]
